# baseline (speedup 1.0000x reference)
import functools

import jax
import jax.numpy as jnp
from jax import lax
from jax.experimental import pallas as pl
from jax.experimental.pallas import tpu as pltpu
from jax.experimental.pallas import tpu_sc as plsc

N = 10000
E = 320000
DIN = 128
DHID = 128
DOUT = 64

NC = 2
NS = 16
NW = NC * NS
NPAD = 10240
RPS = NPAD // NS
CHUNK = 128
CPW0 = 55
CPW1 = 103
CPWM = CPW1
EPAD = NS * (CPW0 + CPW1) * CHUNK
D1 = DHID + 8
D2 = DOUT

_MESH = plsc.VectorSubcoreMesh(core_axis_name="c", subcore_axis_name="s")


def _make_sc_agg(D):
    @functools.partial(
        pl.kernel,
        out_type=jax.ShapeDtypeStruct((NC * NPAD, D), jnp.float32),
        mesh=_MESH,
        compiler_params=pltpu.CompilerParams(use_tc_tiling_on_sc=False),
        scratch_types=[
            pltpu.VMEM((CPWM, CHUNK), jnp.int32),
            pltpu.VMEM((CPWM, CHUNK), jnp.int32),
            pltpu.VMEM((CHUNK, D), jnp.float32),
            pltpu.VMEM_SHARED((NPAD, D), jnp.float32),
            pltpu.SemaphoreType.DMA,
        ],
    )
    def sc_agg(q_hbm, src_hbm, dst_hbm, zeros_hbm, out_hbm,
               src_v, dst_v, rows_v, acc_s, sem):
        c = lax.axis_index("c")
        s = lax.axis_index("s")
        wid = s * NC + c

        pltpu.sync_copy(src_hbm.at[wid], src_v)
        pltpu.sync_copy(dst_hbm.at[wid], dst_v)

        pltpu.sync_copy(zeros_hbm, rows_v)

        def zbody(k, carry):
            pltpu.sync_copy(rows_v, acc_s.at[pl.ds(s * RPS + k * CHUNK, CHUNK)])
            return carry

        lax.fori_loop(0, RPS // CHUNK, zbody, 0)
        plsc.subcore_barrier()

        def body(j, carry):
            pltpu.async_copy(q_hbm.at[src_v.at[j]], rows_v, sem).wait()
            pltpu.sync_copy(rows_v, acc_s.at[dst_v.at[j]], add=True)
            return carry

        cpw_c = jnp.where(c == 0, CPW0, CPW1)
        lax.fori_loop(0, cpw_c, body, 0)
        plsc.subcore_barrier()

        def obody(k, carry):
            base = s * RPS + k * CHUNK
            pltpu.sync_copy(acc_s.at[pl.ds(base, CHUNK)], rows_v)
            pltpu.sync_copy(rows_v, out_hbm.at[pl.ds(c * NPAD + base, CHUNK)])
            return carry

        lax.fori_loop(0, RPS // CHUNK, obody, 0)

    return sc_agg


_sc_agg_d1 = _make_sc_agg(D1)
_sc_agg_d2 = _make_sc_agg(D2)

R = 1024
G = NPAD // R


def _tc_stage1(xp, Wl1, Wr1, b1):
    def body(x_ref, wl_ref, wr_ref, b_ref, qext_ref, r_ref):
        xb = x_ref[...]
        q = jnp.dot(xb, wl_ref[...], preferred_element_type=jnp.float32)
        col = lax.broadcasted_iota(jnp.int32, (R, D1 - DHID), 1)
        ext = jnp.where(col == 0, 1.0, 0.0).astype(jnp.float32)
        qext_ref[...] = jnp.concatenate([q, ext], axis=1)
        r_ref[...] = (
            jnp.dot(xb, wr_ref[...], preferred_element_type=jnp.float32)
            + b_ref[...]
        )

    return pl.pallas_call(
        body,
        grid=(G,),
        in_specs=[
            pl.BlockSpec((R, DIN), lambda i: (i, 0)),
            pl.BlockSpec((DIN, DHID), lambda i: (0, 0)),
            pl.BlockSpec((DIN, DHID), lambda i: (0, 0)),
            pl.BlockSpec((1, DHID), lambda i: (0, 0)),
        ],
        out_specs=[
            pl.BlockSpec((R, D1), lambda i: (i, 0)),
            pl.BlockSpec((R, DHID), lambda i: (i, 0)),
        ],
        out_shape=[
            jax.ShapeDtypeStruct((NPAD, D1), jnp.float32),
            jax.ShapeDtypeStruct((NPAD, DHID), jnp.float32),
        ],
    )(xp, Wl1, Wr1, b1)


def _tc_stage2(agg1, r1, Wl2, Wr2, b2):
    def body(a0_ref, a1_ref, r1_ref, wl_ref, wr_ref, b_ref,
             q2_ref, r2_ref, inv_ref):
        a = a0_ref[...] + a1_ref[...]
        cnt = a[:, DHID:DHID + 1]
        inv = 1.0 / jnp.maximum(cnt, 1.0)
        h = jnp.maximum(a[:, :DHID] * inv + r1_ref[...], 0.0)
        q2_ref[...] = jnp.dot(h, wl_ref[...], preferred_element_type=jnp.float32)
        r2_ref[...] = (
            jnp.dot(h, wr_ref[...], preferred_element_type=jnp.float32)
            + b_ref[...]
        )
        inv_ref[...] = jnp.broadcast_to(inv, (R, DHID))

    return pl.pallas_call(
        body,
        grid=(G,),
        in_specs=[
            pl.BlockSpec((R, D1), lambda i: (i, 0)),
            pl.BlockSpec((R, D1), lambda i: (i + G, 0)),
            pl.BlockSpec((R, DHID), lambda i: (i, 0)),
            pl.BlockSpec((DHID, DOUT), lambda i: (0, 0)),
            pl.BlockSpec((DHID, DOUT), lambda i: (0, 0)),
            pl.BlockSpec((1, DOUT), lambda i: (0, 0)),
        ],
        out_specs=[
            pl.BlockSpec((R, DOUT), lambda i: (i, 0)),
            pl.BlockSpec((R, DOUT), lambda i: (i, 0)),
            pl.BlockSpec((R, DHID), lambda i: (i, 0)),
        ],
        out_shape=[
            jax.ShapeDtypeStruct((NPAD, DOUT), jnp.float32),
            jax.ShapeDtypeStruct((NPAD, DOUT), jnp.float32),
            jax.ShapeDtypeStruct((NPAD, DHID), jnp.float32),
        ],
    )(agg1, agg1, r1, Wl2, Wr2, b2)


def _tc_stage3(agg2, inv, r2):
    def body(a0_ref, a1_ref, inv_ref, r2_ref, z_ref):
        agg = a0_ref[...] + a1_ref[...]
        z_ref[...] = agg * inv_ref[:, 0:1] + r2_ref[...]

    return pl.pallas_call(
        body,
        grid=(G,),
        in_specs=[
            pl.BlockSpec((R, DOUT), lambda i: (i, 0)),
            pl.BlockSpec((R, DOUT), lambda i: (i + G, 0)),
            pl.BlockSpec((R, DHID), lambda i: (i, 0)),
            pl.BlockSpec((R, DOUT), lambda i: (i, 0)),
        ],
        out_specs=pl.BlockSpec((R, DOUT), lambda i: (i, 0)),
        out_shape=jax.ShapeDtypeStruct((NPAD, DOUT), jnp.float32),
    )(agg2, agg2, inv, r2)


def kernel(x, edge_index, Wl1, Wr1, b1, Wl2, Wr2, b2):
    src = edge_index[0].astype(jnp.int32)
    dst = edge_index[1].astype(jnp.int32)
    ea = NS * CPW0 * CHUNK
    srcp = jnp.concatenate([src, jnp.zeros((EPAD - E,), jnp.int32)])
    dstp = jnp.concatenate([dst, jnp.full((EPAD - E,), N, jnp.int32)])

    def _split(flat, fill):
        a = flat[:ea].reshape(NS, CPW0, CHUNK)
        a = jnp.concatenate(
            [a, jnp.full((NS, CPWM - CPW0, CHUNK), fill, jnp.int32)], axis=1)
        b = flat[ea:].reshape(NS, CPW1, CHUNK)
        return jnp.stack([a, b], axis=1).reshape(NW, CPWM, CHUNK)

    src3 = _split(srcp, 0)
    dst3 = _split(dstp, N)
    xp = jnp.zeros((NPAD, DIN), jnp.float32).at[:N].set(x)
    zeros1 = jnp.zeros((CHUNK, D1), jnp.float32)
    zeros2 = jnp.zeros((CHUNK, D2), jnp.float32)

    qext, r1 = _tc_stage1(xp, Wl1, Wr1, b1.reshape(1, DHID))
    agg1 = _sc_agg_d1(qext, src3, dst3, zeros1)
    q2, r2, inv = _tc_stage2(agg1, r1, Wl2, Wr2, b2.reshape(1, DOUT))
    agg2 = _sc_agg_d2(q2, src3, dst3, zeros2)
    z = _tc_stage3(agg2, inv, r2)
    return z[:N]


# speedup vs baseline: 1.2293x; 1.2293x over previous
import functools

import jax
import jax.numpy as jnp
from jax import lax
from jax.experimental import pallas as pl
from jax.experimental.pallas import tpu as pltpu
from jax.experimental.pallas import tpu_sc as plsc

N = 10000
E = 320000
DIN = 128
DHID = 128
DOUT = 64

NC = 2
NS = 16
NW = NC * NS
NPAD = 10240
RPS = NPAD // NS
CHUNK = 128
CPW0 = 103
CPW1 = 55
CPWM = max(CPW0, CPW1)
EPAD = NS * (CPW0 + CPW1) * CHUNK
D1 = DHID + 8
D2 = DOUT

_MESH = plsc.VectorSubcoreMesh(core_axis_name="c", subcore_axis_name="s")


def _make_sc_agg(D):
    @functools.partial(
        pl.kernel,
        out_type=jax.ShapeDtypeStruct((NC * NPAD, D), jnp.float32),
        mesh=_MESH,
        compiler_params=pltpu.CompilerParams(use_tc_tiling_on_sc=False),
        scratch_types=[
            pltpu.VMEM((CPWM, CHUNK), jnp.int32),
            pltpu.VMEM((CPWM, CHUNK), jnp.int32),
            pltpu.VMEM((CHUNK, D), jnp.float32),
            pltpu.VMEM_SHARED((NPAD, D), jnp.float32),
            pltpu.SemaphoreType.DMA,
        ],
    )
    def sc_agg(q_hbm, src_hbm, dst_hbm, zeros_hbm, out_hbm,
               src_v, dst_v, rows_v, acc_s, sem):
        c = lax.axis_index("c")
        s = lax.axis_index("s")
        wid = s * NC + c

        pltpu.sync_copy(src_hbm.at[wid], src_v)
        pltpu.sync_copy(dst_hbm.at[wid], dst_v)

        pltpu.sync_copy(zeros_hbm, rows_v)

        def zbody(k, carry):
            pltpu.sync_copy(rows_v, acc_s.at[pl.ds(s * RPS + k * CHUNK, CHUNK)])
            return carry

        lax.fori_loop(0, RPS // CHUNK, zbody, 0)
        plsc.subcore_barrier()

        def body(j, carry):
            pltpu.async_copy(q_hbm.at[src_v.at[j]], rows_v, sem).wait()
            pltpu.sync_copy(rows_v, acc_s.at[dst_v.at[j]], add=True)
            return carry

        cpw_c = jnp.where(c == 0, CPW0, CPW1)
        lax.fori_loop(0, cpw_c, body, 0)
        plsc.subcore_barrier()

        def obody(k, carry):
            base = s * RPS + k * CHUNK
            pltpu.sync_copy(acc_s.at[pl.ds(base, CHUNK)], rows_v)
            pltpu.sync_copy(rows_v, out_hbm.at[pl.ds(c * NPAD + base, CHUNK)])
            return carry

        lax.fori_loop(0, RPS // CHUNK, obody, 0)

    return sc_agg


_sc_agg_d1 = _make_sc_agg(D1)
_sc_agg_d2 = _make_sc_agg(D2)

R = 1024
G = NPAD // R


def _tc_stage1(xp, Wl1, Wr1, b1):
    def body(x_ref, wl_ref, wr_ref, b_ref, qext_ref, r_ref):
        xb = x_ref[...]
        q = jnp.dot(xb, wl_ref[...], preferred_element_type=jnp.float32)
        col = lax.broadcasted_iota(jnp.int32, (R, D1 - DHID), 1)
        ext = jnp.where(col == 0, 1.0, 0.0).astype(jnp.float32)
        qext_ref[...] = jnp.concatenate([q, ext], axis=1)
        r_ref[...] = (
            jnp.dot(xb, wr_ref[...], preferred_element_type=jnp.float32)
            + b_ref[...]
        )

    return pl.pallas_call(
        body,
        grid=(G,),
        in_specs=[
            pl.BlockSpec((R, DIN), lambda i: (i, 0)),
            pl.BlockSpec((DIN, DHID), lambda i: (0, 0)),
            pl.BlockSpec((DIN, DHID), lambda i: (0, 0)),
            pl.BlockSpec((1, DHID), lambda i: (0, 0)),
        ],
        out_specs=[
            pl.BlockSpec((R, D1), lambda i: (i, 0)),
            pl.BlockSpec((R, DHID), lambda i: (i, 0)),
        ],
        out_shape=[
            jax.ShapeDtypeStruct((NPAD, D1), jnp.float32),
            jax.ShapeDtypeStruct((NPAD, DHID), jnp.float32),
        ],
    )(xp, Wl1, Wr1, b1)


def _tc_stage2(agg1, r1, Wl2, Wr2, b2):
    def body(a0_ref, a1_ref, r1_ref, wl_ref, wr_ref, b_ref,
             q2_ref, r2_ref, inv_ref):
        a = a0_ref[...] + a1_ref[...]
        cnt = a[:, DHID:DHID + 1]
        inv = 1.0 / jnp.maximum(cnt, 1.0)
        h = jnp.maximum(a[:, :DHID] * inv + r1_ref[...], 0.0)
        q2_ref[...] = jnp.dot(h, wl_ref[...], preferred_element_type=jnp.float32)
        r2_ref[...] = (
            jnp.dot(h, wr_ref[...], preferred_element_type=jnp.float32)
            + b_ref[...]
        )
        inv_ref[...] = jnp.broadcast_to(inv, (R, DHID))

    return pl.pallas_call(
        body,
        grid=(G,),
        in_specs=[
            pl.BlockSpec((R, D1), lambda i: (i, 0)),
            pl.BlockSpec((R, D1), lambda i: (i + G, 0)),
            pl.BlockSpec((R, DHID), lambda i: (i, 0)),
            pl.BlockSpec((DHID, DOUT), lambda i: (0, 0)),
            pl.BlockSpec((DHID, DOUT), lambda i: (0, 0)),
            pl.BlockSpec((1, DOUT), lambda i: (0, 0)),
        ],
        out_specs=[
            pl.BlockSpec((R, DOUT), lambda i: (i, 0)),
            pl.BlockSpec((R, DOUT), lambda i: (i, 0)),
            pl.BlockSpec((R, DHID), lambda i: (i, 0)),
        ],
        out_shape=[
            jax.ShapeDtypeStruct((NPAD, DOUT), jnp.float32),
            jax.ShapeDtypeStruct((NPAD, DOUT), jnp.float32),
            jax.ShapeDtypeStruct((NPAD, DHID), jnp.float32),
        ],
    )(agg1, agg1, r1, Wl2, Wr2, b2)


def _tc_stage3(agg2, inv, r2):
    def body(a0_ref, a1_ref, inv_ref, r2_ref, z_ref):
        agg = a0_ref[...] + a1_ref[...]
        z_ref[...] = agg * inv_ref[:, 0:1] + r2_ref[...]

    return pl.pallas_call(
        body,
        grid=(G,),
        in_specs=[
            pl.BlockSpec((R, DOUT), lambda i: (i, 0)),
            pl.BlockSpec((R, DOUT), lambda i: (i + G, 0)),
            pl.BlockSpec((R, DHID), lambda i: (i, 0)),
            pl.BlockSpec((R, DOUT), lambda i: (i, 0)),
        ],
        out_specs=pl.BlockSpec((R, DOUT), lambda i: (i, 0)),
        out_shape=jax.ShapeDtypeStruct((NPAD, DOUT), jnp.float32),
    )(agg2, agg2, inv, r2)


def kernel(x, edge_index, Wl1, Wr1, b1, Wl2, Wr2, b2):
    src = edge_index[0].astype(jnp.int32)
    dst = edge_index[1].astype(jnp.int32)
    ea = NS * CPW0 * CHUNK
    srcp = jnp.concatenate([src, jnp.zeros((EPAD - E,), jnp.int32)])
    dstp = jnp.concatenate([dst, jnp.full((EPAD - E,), N, jnp.int32)])

    def _split(flat, fill):
        a = flat[:ea].reshape(NS, CPW0, CHUNK)
        if CPW0 < CPWM:
            a = jnp.concatenate(
                [a, jnp.full((NS, CPWM - CPW0, CHUNK), fill, jnp.int32)],
                axis=1)
        b = flat[ea:].reshape(NS, CPW1, CHUNK)
        if CPW1 < CPWM:
            b = jnp.concatenate(
                [b, jnp.full((NS, CPWM - CPW1, CHUNK), fill, jnp.int32)],
                axis=1)
        return jnp.stack([a, b], axis=1).reshape(NW, CPWM, CHUNK)

    src3 = _split(srcp, 0)
    dst3 = _split(dstp, N)
    xp = jnp.zeros((NPAD, DIN), jnp.float32).at[:N].set(x)
    zeros1 = jnp.zeros((CHUNK, D1), jnp.float32)
    zeros2 = jnp.zeros((CHUNK, D2), jnp.float32)

    qext, r1 = _tc_stage1(xp, Wl1, Wr1, b1.reshape(1, DHID))
    agg1 = _sc_agg_d1(qext, src3, dst3, zeros1)
    q2, r2, inv = _tc_stage2(agg1, r1, Wl2, Wr2, b2.reshape(1, DOUT))
    agg2 = _sc_agg_d2(q2, src3, dst3, zeros2)
    z = _tc_stage3(agg2, inv, r2)
    return z[:N]
